# Initial kernel scaffold; baseline (speedup 1.0000x reference)
import jax
import jax.numpy as jnp
from jax.experimental import pallas as pl


def kernel(x, edge_index, edge_attr, batch, atom_params, bond_params, gin_params):
    raise NotImplementedError("write your pallas kernel here")


# speedup vs baseline: 1.5675x; 1.5675x over previous
import functools

import jax
import jax.numpy as jnp
from jax import lax
from jax.experimental import pallas as pl
from jax.experimental.pallas import tpu as pltpu
from jax.experimental.pallas import tpu_sc as plsc

N = 10000
E = 160000
NGRAPH = 128
EMB = 300
NG = 3
HG = 128
F = NG * HG
G = 640
NSUB = 16
EPS = E // 32
CH = 40
NCH = EPS // CH
ZR = 40
NZ = N // ZR
BM = 1000


def _padT(W, dout, din):
    Wt = W.T
    return jnp.pad(Wt, ((0, din - Wt.shape[0]), (0, dout - Wt.shape[1])))


def _padb(b, d):
    return jnp.pad(b, (0, d - b.shape[0])).reshape(1, d)


def _bdot(a, w):
    return jnp.dot(a.astype(jnp.bfloat16), w[...],
                   preferred_element_type=jnp.float32)


def _enc3_body(x_ref, w1, b1, w2, b2, w3, b3, out_ref, *, keep_in):
    xin = jnp.concatenate(
        [x_ref[...], jnp.ones((BM, 512 - keep_in), jnp.float32)], axis=1)
    h = _bdot(xin, w1)
    h = jnp.maximum(h + b1[...], 0.0)
    h = _bdot(h, w2)
    h = jnp.maximum(h + b2[...], 0.0)
    h = _bdot(h, w3)
    h = jnp.maximum(h + b3[...], 0.0)
    for g in range(NG):
        out_ref[g] = h[:, g * HG:(g + 1) * HG]


def _encoder(x, params, keep_in):
    (W1, b1), (W2, b2), (W3, b3) = params
    w1 = _padT(W1, F, 512).astype(jnp.bfloat16)
    eb1 = _padb(b1, F)
    w2 = _padT(W2, F, F).astype(jnp.bfloat16)
    eb2 = _padb(b2, F)
    w3 = _padT(W3, F, F).astype(jnp.bfloat16)
    eb3 = _padb(b3, F)
    M = x.shape[0]
    nb = M // BM
    wspec = lambda shp: pl.BlockSpec(shp, lambda i: (0, 0))
    return pl.pallas_call(
        functools.partial(_enc3_body, keep_in=keep_in),
        grid=(nb,),
        in_specs=[
            pl.BlockSpec((BM, keep_in), lambda i: (i, 0)),
            wspec((512, F)), wspec((1, F)),
            wspec((F, F)), wspec((1, F)),
            wspec((F, F)), wspec((1, F)),
        ],
        out_specs=pl.BlockSpec((NG, BM, HG), lambda i: (0, i, 0)),
        out_shape=jax.ShapeDtypeStruct((NG, M, HG), jnp.float32),
    )(x, w1, eb1, w2, eb2, w3, eb3)


def _sc_agg_body(xflat, eaflat, src_h, dst_h, out, src_v, dst_v, xr, ea_v,
                 zbuf, acc, sem):
    c = lax.axis_index("c")
    s = lax.axis_index("s")
    w = c * NSUB + s

    pltpu.sync_copy(dst_h.at[w], dst_v)

    def zrow(r, carry):
        for k in range(HG // 16):
            zbuf[r, pl.ds(k * 16, 16)] = jnp.zeros((16,), jnp.float32)
        return carry
    lax.fori_loop(0, ZR, zrow, 0)

    for g in range(NG):
        pltpu.sync_copy(src_h.at[g * 32 + w], src_v)
        for t in range((NZ + NSUB - 1) // NSUB):
            idx = s + NSUB * t

            @pl.when(idx < NZ)
            def _():
                pltpu.sync_copy(zbuf, acc.at[pl.ds(idx * ZR, ZR)])
        plsc.subcore_barrier()

        def chunk(j, carry):
            pltpu.async_copy(xflat.at[src_v.at[j]], xr, sem).wait()
            pltpu.sync_copy(
                eaflat.at[pl.ds(g * E + w * EPS + j * CH, CH)], ea_v)

            def row(r, carry2):
                for k in range(HG // 16):
                    sl = pl.ds(k * 16, 16)
                    xr[r, sl] = jnp.maximum(xr[r, sl] + ea_v[r, sl], 0.0)
                return carry2
            lax.fori_loop(0, CH, row, 0)
            pltpu.sync_copy(xr, acc.at[dst_v.at[j]], add=True)
            return carry
        lax.fori_loop(0, NCH, chunk, 0)
        plsc.subcore_barrier()

        for t in range((NZ + NSUB - 1) // NSUB):
            idx = s + NSUB * t

            @pl.when(idx < NZ)
            def _():
                pltpu.sync_copy(acc.at[pl.ds(idx * ZR, ZR)], ea_v)
                pltpu.sync_copy(
                    ea_v, out.at[pl.ds((c * NG + g) * N + idx * ZR, ZR)])


def _make_sc_agg():
    mesh = plsc.VectorSubcoreMesh(core_axis_name="c", subcore_axis_name="s",
                                  num_cores=2, num_subcores=NSUB)
    return pl.kernel(
        _sc_agg_body,
        out_type=jax.ShapeDtypeStruct((2 * NG * N, HG), jnp.float32),
        mesh=mesh,
        scratch_types=[
            pltpu.VMEM((NCH, CH), jnp.int32),
            pltpu.VMEM((NCH, CH), jnp.int32),
            pltpu.VMEM((CH, HG), jnp.float32),
            pltpu.VMEM((CH, HG), jnp.float32),
            pltpu.VMEM((ZR, HG), jnp.float32),
            pltpu.VMEM_SHARED((N, HG), jnp.float32),
            pltpu.SemaphoreType.DMA,
        ],
    )


def _ginA_body(x3, agg, w1, b1, h_out):
    xa = jnp.concatenate(
        [x3[g] + agg[0, g] + agg[1, g] for g in range(NG)], axis=1)
    h = _bdot(xa, w1) + b1[...]
    h_out[...] = h


def _ginB_body(h1, mv1, w2, b2, h_out):
    mean = mv1[0, :]
    istd = mv1[1, :]
    y = jnp.maximum((h1[...] - mean[None, :]) * istd[None, :], 0.0)
    h = _bdot(y[:, :600], w2) + b2[...]
    h_out[...] = h


def _ginC_body(h2, mv2, x3_out, xc_out, *, relu):
    mean = mv2[0, :]
    istd = mv2[1, :]
    y = (h2[...] - mean[None, :]) * istd[None, :]
    if relu:
        y = jnp.maximum(y, 0.0)
    for g in range(NG):
        x3_out[g] = y[:, g * HG:(g + 1) * HG]
    xc_out[...] = y


def _gin_dense(x3, agg, p):
    nb = N // BM
    w1 = _padT(p["W1"], G, F).astype(jnp.bfloat16)
    b1 = _padb(p["b1"], G)
    w2 = _padT(p["W2"], F, 600).astype(jnp.bfloat16)
    b2 = _padb(p["b2"], F)
    agg4 = agg.reshape(2, NG, N, HG)
    wspec = lambda shp: pl.BlockSpec(shp, lambda i: (0, 0))
    h1 = pl.pallas_call(
        _ginA_body,
        grid=(nb,),
        in_specs=[
            pl.BlockSpec((NG, BM, HG), lambda i: (0, i, 0)),
            pl.BlockSpec((2, NG, BM, HG), lambda i: (0, 0, i, 0)),
            wspec((F, G)), wspec((1, G)),
        ],
        out_specs=pl.BlockSpec((BM, G), lambda i: (i, 0)),
        out_shape=jax.ShapeDtypeStruct((N, G), jnp.float32),
    )(x3, agg4, w1, b1)
    mv1 = jnp.stack([jnp.mean(h1, axis=0),
                     lax.rsqrt(jnp.var(h1, axis=0) + 1e-5)])
    h2 = pl.pallas_call(
        _ginB_body,
        grid=(nb,),
        in_specs=[
            pl.BlockSpec((BM, G), lambda i: (i, 0)),
            wspec((2, G)),
            wspec((600, F)), wspec((1, F)),
        ],
        out_specs=pl.BlockSpec((BM, F), lambda i: (i, 0)),
        out_shape=jax.ShapeDtypeStruct((N, F), jnp.float32),
    )(h1, mv1, w2, b2)
    mv2 = jnp.stack([jnp.mean(h2, axis=0),
                     lax.rsqrt(jnp.var(h2, axis=0) + 1e-5)])
    return h2, mv2


def _gin_norm(h2, mv2, relu):
    nb = N // BM
    wspec = lambda shp: pl.BlockSpec(shp, lambda i: (0, 0))
    x3, xc = pl.pallas_call(
        functools.partial(_ginC_body, relu=relu),
        grid=(nb,),
        in_specs=[pl.BlockSpec((BM, F), lambda i: (i, 0)), wspec((2, F))],
        out_specs=[pl.BlockSpec((NG, BM, HG), lambda i: (0, i, 0)),
                   pl.BlockSpec((BM, F), lambda i: (i, 0))],
        out_shape=[jax.ShapeDtypeStruct((NG, N, HG), jnp.float32),
                   jax.ShapeDtypeStruct((N, F), jnp.float32)],
    )(h2, mv2)
    return x3, xc


def _pool_body(x3, batch3, out):
    xcat = jnp.concatenate([x3[g] for g in range(NG)], axis=1)
    b = batch3[0, 0, :]
    onehot = (b[:, None] == lax.broadcasted_iota(jnp.int32, (BM, NGRAPH), 1))
    onehot = onehot.astype(jnp.float32)
    acc = lax.dot_general(onehot, xcat, (((0,), (0,)), ((), ())),
                          preferred_element_type=jnp.float32,
                          precision=lax.Precision.HIGHEST)

    @pl.when(pl.program_id(0) == 0)
    def _():
        out[...] = jnp.zeros_like(out)
    out[...] += acc


def _pool(x3, batch):
    nb = N // BM
    batch3 = batch.reshape(nb, 1, BM)
    return pl.pallas_call(
        _pool_body,
        grid=(nb,),
        in_specs=[pl.BlockSpec((NG, BM, HG), lambda i: (0, i, 0)),
                  pl.BlockSpec((1, 1, BM), lambda i: (i, 0, 0))],
        out_specs=pl.BlockSpec((NGRAPH, F), lambda i: (0, 0)),
        out_shape=jax.ShapeDtypeStruct((NGRAPH, F), jnp.float32),
    )(x3, batch3)


def kernel(x, edge_index, edge_attr, batch, atom_params, bond_params,
           gin_params):
    x3 = _encoder(x.astype(jnp.float32), atom_params, 128)

    order = jnp.argsort(edge_index[1])
    i = jnp.arange(E)
    perm = order[(i // CH) + (E // CH) * (i % CH)]
    src = edge_index[0][perm]
    dst = edge_index[1][perm]
    edge_attr = edge_attr[perm]
    src_h = jnp.stack([src, src + N, src + 2 * N]).reshape(NG * 32, NCH, CH)
    dst_h = dst.reshape(32, NCH, CH)
    ea3 = _encoder(edge_attr.astype(jnp.float32), bond_params, 16)
    eaflat = ea3.reshape(NG * E, HG)

    sc_agg = _make_sc_agg()

    xc = None
    for i in range(len(gin_params)):
        xflat = x3.reshape(NG * N, HG)
        agg = sc_agg(xflat, eaflat, src_h, dst_h)
        h2, mv2 = _gin_dense(x3, agg, gin_params[i])
        x3, xc = _gin_norm(h2, mv2, relu=(i != len(gin_params) - 1))

    xpool = _pool(x3, batch)
    return (xpool[:, :EMB], xc[:, :EMB])
